# baseline (speedup 1.0000x reference)
import jax
import jax.numpy as jnp
from jax import lax
from jax.experimental import pallas as pl
from jax.experimental.pallas import tpu as pltpu
from jax.experimental.pallas import tpu_sc as plsc

N_NODES = 10000
N_EDGES = 320000
D = 128
NPAD = 10240
NC = 2
NS = 16
NW = NC * NS
NB = 80
CHB = 16
NCH = NB // CHB
EPT = NB * 128
E_PAD = NW * EPT
RPT = NPAD // NS
BLK = 1280
GRID = NPAD // BLK


def _sc_mesh():
    return plsc.VectorSubcoreMesh(core_axis_name="c", subcore_axis_name="s")


_SC_PARAMS = pltpu.CompilerParams(needs_layout_passes=False)


def _deg_body(src_hbm, dst_hbm, z_hbm, out_hbm, sidx, didx, hs, hd):
    c = lax.axis_index("c")
    s = lax.axis_index("s")
    w = s * NC + c
    pltpu.sync_copy(src_hbm.at[w], sidx)
    pltpu.sync_copy(dst_hbm.at[w], didx)
    pltpu.sync_copy(z_hbm, hs)
    pltpu.sync_copy(z_hbm, hd)
    ones = jnp.ones((16,), jnp.float32)

    def body(i, carry):
        si = sidx[pl.ds(i * 16, 16)]
        di = didx[pl.ds(i * 16, 16)]
        plsc.addupdate_scatter(hs, [si], ones)
        plsc.addupdate_scatter(hd, [di], ones)
        return carry

    lax.fori_loop(0, EPT // 16, body, 0)
    pltpu.sync_copy(hs, out_hbm.at[0, w])
    pltpu.sync_copy(hd, out_hbm.at[1, w])


_deg = pl.kernel(
    _deg_body,
    out_type=jax.ShapeDtypeStruct((2, NW, NPAD), jnp.float32),
    mesh=_sc_mesh(),
    scratch_types=[
        pltpu.VMEM((EPT,), jnp.int32),
        pltpu.VMEM((EPT,), jnp.int32),
        pltpu.VMEM((NPAD,), jnp.float32),
        pltpu.VMEM((NPAD,), jnp.float32),
    ],
    compiler_params=_SC_PARAMS,
)


def _seg_body(tab_hbm, src_hbm, dst_hbm, z_hbm, out_hbm, sidx, didx, rows, acc):
    c = lax.axis_index("c")
    s = lax.axis_index("s")
    w = s * NC + c
    pltpu.sync_copy(src_hbm.at[w], sidx)
    pltpu.sync_copy(dst_hbm.at[w], didx)
    base = s * RPT
    for k in range(RPT // 128):
        pltpu.sync_copy(z_hbm, acc.at[pl.ds(base + k * 128, 128)])
    plsc.subcore_barrier()

    def body(b, carry):
        pltpu.sync_copy(tab_hbm.at[sidx.at[b]], rows)
        pltpu.sync_copy(rows, acc.at[didx.at[b]], add=True)
        return carry

    lax.fori_loop(0, NB, body, 0)
    plsc.subcore_barrier()
    pltpu.sync_copy(acc.at[pl.ds(base, RPT)], out_hbm.at[c, pl.ds(base, RPT)])


_segsum = pl.kernel(
    _seg_body,
    out_type=jax.ShapeDtypeStruct((NC, NPAD, D), jnp.float32),
    mesh=_sc_mesh(),
    scratch_types=[
        pltpu.VMEM((NB, 128), jnp.int32),
        pltpu.VMEM((NB, 128), jnp.int32),
        pltpu.VMEM((128, D), jnp.float32),
        pltpu.VMEM_SHARED((NPAD, D), jnp.float32),
    ],
    compiler_params=_SC_PARAMS,
)


def _col_bcast(deg_nw_blk):
    return lax.dot_general(
        deg_nw_blk, jnp.ones((NW, 128), jnp.float32),
        (((0,), (0,)), ((), ())), preferred_element_type=jnp.float32)


def _rsq(counts):
    return lax.rsqrt(jnp.maximum(counts, 1.0))


def _l1_body(x_ref, w_ref, dg_ref, t0_ref):
    rout = _rsq(_col_bcast(dg_ref[0]))
    t0_ref[...] = jnp.dot(x_ref[...] * rout, w_ref[...],
                          preferred_element_type=jnp.float32)


_layer1 = pl.pallas_call(
    _l1_body,
    grid=(GRID,),
    in_specs=[
        pl.BlockSpec((BLK, D), lambda i: (i, 0)),
        pl.BlockSpec((D, D), lambda i: (0, 0)),
        pl.BlockSpec((1, NW, BLK), lambda i: (0, 0, i)),
    ],
    out_specs=pl.BlockSpec((BLK, D), lambda i: (i, 0)),
    out_shape=jax.ShapeDtypeStruct((NPAD, D), jnp.float32),
)


def _l2_body(p_ref, dg_ref, b0_ref, w_ref, t1_ref):
    agg = p_ref[0] + p_ref[1]
    rin = _rsq(_col_bcast(dg_ref[1]))
    rout = _rsq(_col_bcast(dg_ref[0]))
    h = jnp.maximum(agg * rin + b0_ref[...], 0.0)
    t1_ref[...] = jnp.dot(h * rout, w_ref[...],
                          preferred_element_type=jnp.float32)


_layer2 = pl.pallas_call(
    _l2_body,
    grid=(GRID,),
    in_specs=[
        pl.BlockSpec((NC, BLK, D), lambda i: (0, i, 0)),
        pl.BlockSpec((2, NW, BLK), lambda i: (0, 0, i)),
        pl.BlockSpec((1, D), lambda i: (0, 0)),
        pl.BlockSpec((D, D), lambda i: (0, 0)),
    ],
    out_specs=pl.BlockSpec((BLK, D), lambda i: (i, 0)),
    out_shape=jax.ShapeDtypeStruct((NPAD, D), jnp.float32),
)


def _fin_body(p_ref, dg_ref, b1_ref, o_ref, acc_ref, sn_ref):
    i = pl.program_id(0)

    @pl.when(i == 0)
    def _init():
        acc_ref[...] = jnp.zeros_like(acc_ref)
        sn_ref[0] = 0.0

    agg = p_ref[0] + p_ref[1]
    rin = _rsq(_col_bcast(dg_ref[0]))
    h = agg * rin + b1_ref[...]
    rowid = lax.broadcasted_iota(jnp.int32, (BLK, D), 0) + i * BLK
    h = jnp.where(rowid < N_NODES, h, 0.0)
    acc_ref[...] += jnp.sum(h, axis=0, keepdims=True)
    sn_ref[0] += jnp.sum(jnp.sqrt(jnp.sum(h * h, axis=1)))

    @pl.when(i == pl.num_programs(0) - 1)
    def _done():
        factor = jnp.sqrt(jnp.float32(D)) * (jnp.float32(N_NODES) / sn_ref[0])
        o_ref[...] = acc_ref[...] * factor


_final = pl.pallas_call(
    _fin_body,
    grid=(GRID,),
    in_specs=[
        pl.BlockSpec((NC, BLK, D), lambda i: (0, i, 0)),
        pl.BlockSpec((1, NW, BLK), lambda i: (1, 0, i)),
        pl.BlockSpec((1, D), lambda i: (0, 0)),
    ],
    out_specs=pl.BlockSpec((1, D), lambda i: (0, 0)),
    out_shape=jax.ShapeDtypeStruct((1, D), jnp.float32),
    scratch_shapes=[
        pltpu.VMEM((1, D), jnp.float32),
        pltpu.SMEM((1,), jnp.float32),
    ],
)


def kernel(x, edge_index, edge_attr, W0, b0, W1, b1):
    src = edge_index[0]
    dst = edge_index[1]
    padv = jnp.full((E_PAD - N_EDGES,), N_NODES, jnp.int32)
    srcp = jnp.concatenate([src, padv])
    dstp = jnp.concatenate([dst, padv])
    src1 = srcp.reshape(NW, EPT)
    dst1 = dstp.reshape(NW, EPT)
    src3 = srcp.reshape(NW, NB, 128)
    dst3 = dstp.reshape(NW, NB, 128)
    x_pad = jnp.zeros((NPAD, D), jnp.float32).at[:N_NODES].set(x)
    z1 = jnp.zeros((NPAD,), jnp.float32)
    z2 = jnp.zeros((128, D), jnp.float32)
    b0r = b0.reshape(1, D)
    b1r = b1.reshape(1, D)

    degp = _deg(src1, dst1, z1)
    t0 = _layer1(x_pad, W0, degp)
    p0 = _segsum(t0, src3, dst3, z2)
    t1 = _layer2(p0, degp, b0r, W1)
    p1 = _segsum(t1, src3, dst3, z2)
    return _final(p1, degp, b1r)


# speedup vs baseline: 2.6354x; 2.6354x over previous
import jax
import jax.numpy as jnp
from jax import lax
from jax.experimental import pallas as pl
from jax.experimental.pallas import tpu as pltpu
from jax.experimental.pallas import tpu_sc as plsc

N_NODES = 10000
N_EDGES = 320000
D = 128
NPAD = 10240
NC = 2
NS = 16
NW = NC * NS
NB = 80
CHB = 16
NCH = NB // CHB
EPT = NB * 128
E_PAD = NW * EPT
RPT = NPAD // NS
BLK = 1280
GRID = NPAD // BLK


def _sc_mesh():
    return plsc.VectorSubcoreMesh(core_axis_name="c", subcore_axis_name="s")


_SC_PARAMS = pltpu.CompilerParams(needs_layout_passes=False)


def _deg_body(src_hbm, dst_hbm, z_hbm, out_hbm, sidx, didx, hs, hd):
    c = lax.axis_index("c")
    s = lax.axis_index("s")
    w = s * NC + c
    pltpu.sync_copy(src_hbm.at[w], sidx)
    pltpu.sync_copy(dst_hbm.at[w], didx)
    pltpu.sync_copy(z_hbm, hs)
    pltpu.sync_copy(z_hbm, hd)
    ones = jnp.ones((16,), jnp.float32)

    def body(i, carry):
        si = sidx[pl.ds(i * 16, 16)]
        di = didx[pl.ds(i * 16, 16)]
        plsc.addupdate_scatter(hs, [si], ones)
        plsc.addupdate_scatter(hd, [di], ones)
        return carry

    lax.fori_loop(0, EPT // 16, body, 0)
    pltpu.sync_copy(hs, out_hbm.at[0, w])
    pltpu.sync_copy(hd, out_hbm.at[1, w])


_deg = pl.kernel(
    _deg_body,
    out_type=jax.ShapeDtypeStruct((2, NW, NPAD), jnp.float32),
    mesh=_sc_mesh(),
    scratch_types=[
        pltpu.VMEM((EPT,), jnp.int32),
        pltpu.VMEM((EPT,), jnp.int32),
        pltpu.VMEM((NPAD,), jnp.float32),
        pltpu.VMEM((NPAD,), jnp.float32),
    ],
    compiler_params=_SC_PARAMS,
)


def _seg_body(tab_hbm, src_hbm, dst_hbm, z_hbm, out_hbm, sidx, didx, rows, acc):
    c = lax.axis_index("c")
    s = lax.axis_index("s")
    w = s * NC + c
    pltpu.sync_copy(src_hbm.at[w], sidx)
    pltpu.sync_copy(dst_hbm.at[w], didx)
    base = s * RPT
    for k in range(RPT // 128):
        pltpu.sync_copy(z_hbm, acc.at[pl.ds(base + k * 128, 128)])
    plsc.subcore_barrier()

    def body(b, carry):
        pltpu.sync_copy(tab_hbm.at[sidx.at[b]], rows)
        pltpu.sync_copy(rows, acc.at[didx.at[b]], add=True)
        return carry

    lax.fori_loop(0, NB, body, 0)
    plsc.subcore_barrier()
    pltpu.sync_copy(acc.at[pl.ds(base, RPT)], out_hbm.at[c, pl.ds(base, RPT)])


_segsum = pl.kernel(
    _seg_body,
    out_type=jax.ShapeDtypeStruct((NC, NPAD, D), jnp.float32),
    mesh=_sc_mesh(),
    scratch_types=[
        pltpu.VMEM((NB, 128), jnp.int32),
        pltpu.VMEM((NB, 128), jnp.int32),
        pltpu.VMEM((128, D), jnp.float32),
        pltpu.VMEM_SHARED((NPAD, D), jnp.float32),
    ],
    compiler_params=_SC_PARAMS,
)


def _col_bcast(deg_nw_blk):
    return lax.dot_general(
        deg_nw_blk, jnp.ones((NW, 128), jnp.float32),
        (((0,), (0,)), ((), ())), preferred_element_type=jnp.float32)


def _rsq(counts):
    return lax.rsqrt(jnp.maximum(counts, 1.0))


def _l1_body(x_ref, w_ref, dg_ref, t0_ref):
    rout = _rsq(_col_bcast(dg_ref[0]))
    t0_ref[...] = jnp.dot(x_ref[...] * rout, w_ref[...],
                          preferred_element_type=jnp.float32)


_layer1 = pl.pallas_call(
    _l1_body,
    grid=(GRID,),
    in_specs=[
        pl.BlockSpec((BLK, D), lambda i: (i, 0)),
        pl.BlockSpec((D, D), lambda i: (0, 0)),
        pl.BlockSpec((1, NW, BLK), lambda i: (0, 0, i)),
    ],
    out_specs=pl.BlockSpec((BLK, D), lambda i: (i, 0)),
    out_shape=jax.ShapeDtypeStruct((NPAD, D), jnp.float32),
)


def _l2_body(p_ref, dg_ref, b0_ref, w_ref, t1_ref):
    agg = p_ref[0] + p_ref[1]
    rin = _rsq(_col_bcast(dg_ref[1]))
    rout = _rsq(_col_bcast(dg_ref[0]))
    h = jnp.maximum(agg * rin + b0_ref[...], 0.0)
    t1_ref[...] = jnp.dot(h * rout, w_ref[...],
                          preferred_element_type=jnp.float32)


_layer2 = pl.pallas_call(
    _l2_body,
    grid=(GRID,),
    in_specs=[
        pl.BlockSpec((NC, BLK, D), lambda i: (0, i, 0)),
        pl.BlockSpec((2, NW, BLK), lambda i: (0, 0, i)),
        pl.BlockSpec((1, D), lambda i: (0, 0)),
        pl.BlockSpec((D, D), lambda i: (0, 0)),
    ],
    out_specs=pl.BlockSpec((BLK, D), lambda i: (i, 0)),
    out_shape=jax.ShapeDtypeStruct((NPAD, D), jnp.float32),
)


def _fin_body(p_ref, dg_ref, b1_ref, o_ref, acc_ref, sn_ref):
    i = pl.program_id(0)

    @pl.when(i == 0)
    def _init():
        acc_ref[...] = jnp.zeros_like(acc_ref)
        sn_ref[0] = 0.0

    agg = p_ref[0] + p_ref[1]
    rin = _rsq(_col_bcast(dg_ref[0]))
    h = agg * rin + b1_ref[...]
    rowid = lax.broadcasted_iota(jnp.int32, (BLK, D), 0) + i * BLK
    h = jnp.where(rowid < N_NODES, h, 0.0)
    acc_ref[...] += jnp.sum(h, axis=0, keepdims=True)
    sn_ref[0] += jnp.sum(jnp.sqrt(jnp.sum(h * h, axis=1)))

    @pl.when(i == pl.num_programs(0) - 1)
    def _done():
        factor = jnp.sqrt(jnp.float32(D)) * (jnp.float32(N_NODES) / sn_ref[0])
        o_ref[...] = acc_ref[...] * factor


_final = pl.pallas_call(
    _fin_body,
    grid=(GRID,),
    in_specs=[
        pl.BlockSpec((NC, BLK, D), lambda i: (0, i, 0)),
        pl.BlockSpec((1, NW, BLK), lambda i: (1, 0, i)),
        pl.BlockSpec((1, D), lambda i: (0, 0)),
    ],
    out_specs=pl.BlockSpec((1, D), lambda i: (0, 0)),
    out_shape=jax.ShapeDtypeStruct((1, D), jnp.float32),
    scratch_shapes=[
        pltpu.VMEM((1, D), jnp.float32),
        pltpu.SMEM((1,), jnp.float32),
    ],
)


def kernel(x, edge_index, edge_attr, W0, b0, W1, b1):
    src = edge_index[0]
    dst = edge_index[1]
    padv = N_NODES + (jnp.arange(E_PAD - N_EDGES, dtype=jnp.int32)
                      % (NPAD - N_NODES))
    srcp = jnp.concatenate([src, padv])
    dstp = jnp.concatenate([dst, padv])
    src1 = srcp.reshape(NW, EPT)
    dst1 = dstp.reshape(NW, EPT)
    src3 = srcp.reshape(NW, NB, 128)
    dst3 = dstp.reshape(NW, NB, 128)
    x_pad = jnp.zeros((NPAD, D), jnp.float32).at[:N_NODES].set(x)
    z1 = jnp.zeros((NPAD,), jnp.float32)
    z2 = jnp.zeros((128, D), jnp.float32)
    b0r = b0.reshape(1, D)
    b1r = b1.reshape(1, D)

    degp = _deg(src1, dst1, z1)
    t0 = _layer1(x_pad, W0, degp)
    p0 = _segsum(t0, src3, dst3, z2)
    t1 = _layer2(p0, degp, b0r, W1)
    p1 = _segsum(t1, src3, dst3, z2)
    return _final(p1, degp, b1r)


# speedup vs baseline: 3.0960x; 1.1748x over previous
import jax
import jax.numpy as jnp
from jax import lax
from jax.experimental import pallas as pl
from jax.experimental.pallas import tpu as pltpu
from jax.experimental.pallas import tpu_sc as plsc

N_NODES = 10000
N_EDGES = 320000
D = 128
NPAD = 10240
NC = 2
NS = 16
NW = NC * NS
NB = 80
CHB = 16
NCH = NB // CHB
EPT = NB * 128
E_PAD = NW * EPT
RPT = NPAD // NS
BLK = 1280
GRID = NPAD // BLK


def _sc_mesh():
    return plsc.VectorSubcoreMesh(core_axis_name="c", subcore_axis_name="s")


_SC_PARAMS = pltpu.CompilerParams(needs_layout_passes=False)
_SC_PARAMS_BF = pltpu.CompilerParams(needs_layout_passes=False,
                                     use_tc_tiling_on_sc=False)


def _deg_body(src_hbm, dst_hbm, z_hbm, out_hbm, sidx, didx, hs, hd):
    c = lax.axis_index("c")
    s = lax.axis_index("s")
    w = s * NC + c
    pltpu.sync_copy(src_hbm.at[w], sidx)
    pltpu.sync_copy(dst_hbm.at[w], didx)
    pltpu.sync_copy(z_hbm, hs)
    pltpu.sync_copy(z_hbm, hd)
    ones = jnp.ones((16,), jnp.float32)

    def body(i, carry):
        si = sidx[pl.ds(i * 16, 16)]
        di = didx[pl.ds(i * 16, 16)]
        plsc.addupdate_scatter(hs, [si], ones)
        plsc.addupdate_scatter(hd, [di], ones)
        return carry

    lax.fori_loop(0, EPT // 16, body, 0)
    pltpu.sync_copy(hs, out_hbm.at[0, w])
    pltpu.sync_copy(hd, out_hbm.at[1, w])


_deg = pl.kernel(
    _deg_body,
    out_type=jax.ShapeDtypeStruct((2, NW, NPAD), jnp.float32),
    mesh=_sc_mesh(),
    scratch_types=[
        pltpu.VMEM((EPT,), jnp.int32),
        pltpu.VMEM((EPT,), jnp.int32),
        pltpu.VMEM((NPAD,), jnp.float32),
        pltpu.VMEM((NPAD,), jnp.float32),
    ],
    compiler_params=_SC_PARAMS,
)


def _seg_body(tab_hbm, src_hbm, dst_hbm, z_hbm, out_hbm, sidx, didx, rows, acc):
    c = lax.axis_index("c")
    s = lax.axis_index("s")
    w = s * NC + c
    pltpu.sync_copy(src_hbm.at[w], sidx)
    pltpu.sync_copy(dst_hbm.at[w], didx)
    base = s * RPT
    for k in range(RPT // 128):
        pltpu.sync_copy(z_hbm, acc.at[pl.ds(base + k * 128, 128)])
    plsc.subcore_barrier()

    def body(b, carry):
        pltpu.sync_copy(tab_hbm.at[sidx.at[b]], rows)
        pltpu.sync_copy(rows, acc.at[didx.at[b]], add=True)
        return carry

    lax.fori_loop(0, NB, body, 0)
    plsc.subcore_barrier()
    pltpu.sync_copy(acc.at[pl.ds(base, RPT)], out_hbm.at[c, pl.ds(base, RPT)])


_segsum = pl.kernel(
    _seg_body,
    out_type=jax.ShapeDtypeStruct((NC, NPAD, D), jnp.bfloat16),
    mesh=_sc_mesh(),
    scratch_types=[
        pltpu.VMEM((NB, 128), jnp.int32),
        pltpu.VMEM((NB, 128), jnp.int32),
        pltpu.VMEM((128, D), jnp.bfloat16),
        pltpu.VMEM_SHARED((NPAD, D), jnp.bfloat16),
    ],
    compiler_params=_SC_PARAMS_BF,
)


def _col_bcast(deg_nw_blk):
    return lax.dot_general(
        deg_nw_blk, jnp.ones((NW, 128), jnp.float32),
        (((0,), (0,)), ((), ())), preferred_element_type=jnp.float32)


def _rsq(counts):
    return lax.rsqrt(jnp.maximum(counts, 1.0))


def _l1_body(x_ref, w_ref, dg_ref, t0_ref):
    rout = _rsq(_col_bcast(dg_ref[0]))
    t0_ref[...] = jnp.dot(x_ref[...] * rout, w_ref[...],
                          preferred_element_type=jnp.float32
                          ).astype(jnp.bfloat16)


_layer1 = pl.pallas_call(
    _l1_body,
    grid=(GRID,),
    in_specs=[
        pl.BlockSpec((BLK, D), lambda i: (i, 0)),
        pl.BlockSpec((D, D), lambda i: (0, 0)),
        pl.BlockSpec((1, NW, BLK), lambda i: (0, 0, i)),
    ],
    out_specs=pl.BlockSpec((BLK, D), lambda i: (i, 0)),
    out_shape=jax.ShapeDtypeStruct((NPAD, D), jnp.bfloat16),
)


def _l2_body(p_ref, dg_ref, b0_ref, w_ref, t1_ref):
    agg = p_ref[0].astype(jnp.float32) + p_ref[1].astype(jnp.float32)
    rin = _rsq(_col_bcast(dg_ref[1]))
    rout = _rsq(_col_bcast(dg_ref[0]))
    h = jnp.maximum(agg * rin + b0_ref[...], 0.0)
    t1_ref[...] = jnp.dot(h * rout, w_ref[...],
                          preferred_element_type=jnp.float32
                          ).astype(jnp.bfloat16)


_layer2 = pl.pallas_call(
    _l2_body,
    grid=(GRID,),
    in_specs=[
        pl.BlockSpec((NC, BLK, D), lambda i: (0, i, 0)),
        pl.BlockSpec((2, NW, BLK), lambda i: (0, 0, i)),
        pl.BlockSpec((1, D), lambda i: (0, 0)),
        pl.BlockSpec((D, D), lambda i: (0, 0)),
    ],
    out_specs=pl.BlockSpec((BLK, D), lambda i: (i, 0)),
    out_shape=jax.ShapeDtypeStruct((NPAD, D), jnp.bfloat16),
)


def _fin_body(p_ref, dg_ref, b1_ref, o_ref, acc_ref, sn_ref):
    i = pl.program_id(0)

    @pl.when(i == 0)
    def _init():
        acc_ref[...] = jnp.zeros_like(acc_ref)
        sn_ref[0] = 0.0

    agg = p_ref[0].astype(jnp.float32) + p_ref[1].astype(jnp.float32)
    rin = _rsq(_col_bcast(dg_ref[0]))
    h = agg * rin + b1_ref[...]
    rowid = lax.broadcasted_iota(jnp.int32, (BLK, D), 0) + i * BLK
    h = jnp.where(rowid < N_NODES, h, 0.0)
    acc_ref[...] += jnp.sum(h, axis=0, keepdims=True)
    sn_ref[0] += jnp.sum(jnp.sqrt(jnp.sum(h * h, axis=1)))

    @pl.when(i == pl.num_programs(0) - 1)
    def _done():
        factor = jnp.sqrt(jnp.float32(D)) * (jnp.float32(N_NODES) / sn_ref[0])
        o_ref[...] = acc_ref[...] * factor


_final = pl.pallas_call(
    _fin_body,
    grid=(GRID,),
    in_specs=[
        pl.BlockSpec((NC, BLK, D), lambda i: (0, i, 0)),
        pl.BlockSpec((1, NW, BLK), lambda i: (1, 0, i)),
        pl.BlockSpec((1, D), lambda i: (0, 0)),
    ],
    out_specs=pl.BlockSpec((1, D), lambda i: (0, 0)),
    out_shape=jax.ShapeDtypeStruct((1, D), jnp.float32),
    scratch_shapes=[
        pltpu.VMEM((1, D), jnp.float32),
        pltpu.SMEM((1,), jnp.float32),
    ],
)


def kernel(x, edge_index, edge_attr, W0, b0, W1, b1):
    src = edge_index[0]
    dst = edge_index[1]
    padv = N_NODES + (jnp.arange(E_PAD - N_EDGES, dtype=jnp.int32)
                      % (NPAD - N_NODES))
    srcp = jnp.concatenate([src, padv])
    dstp = jnp.concatenate([dst, padv])
    src1 = srcp.reshape(NW, EPT)
    dst1 = dstp.reshape(NW, EPT)
    src3 = srcp.reshape(NW, NB, 128)
    dst3 = dstp.reshape(NW, NB, 128)
    x_pad = jnp.zeros((NPAD, D), jnp.float32).at[:N_NODES].set(x)
    z1 = jnp.zeros((NPAD,), jnp.float32)
    z2 = jnp.zeros((128, D), jnp.bfloat16)
    b0r = b0.reshape(1, D)
    b1r = b1.reshape(1, D)

    degp = _deg(src1, dst1, z1)
    t0 = _layer1(x_pad, W0, degp)
    p0 = _segsum(t0, src3, dst3, z2)
    t1 = _layer2(p0, degp, b0r, W1)
    p1 = _segsum(t1, src3, dst3, z2)
    return _final(p1, degp, b1r)


# speedup vs baseline: 3.6103x; 1.1661x over previous
import jax
import jax.numpy as jnp
from jax import lax
from jax.experimental import pallas as pl
from jax.experimental.pallas import tpu as pltpu
from jax.experimental.pallas import tpu_sc as plsc

N_NODES = 10000
N_EDGES = 320000
D = 128
NPAD = 10240
NC = 2
NS = 16
NW = NC * NS
NB = 80
EPT = NB * 128
E_PAD = NW * EPT
RPT = NPAD // NS
BLK = 1280
GRID = NPAD // BLK


def _sc_mesh():
    return plsc.VectorSubcoreMesh(core_axis_name="c", subcore_axis_name="s")


_SC_PARAMS = pltpu.CompilerParams(needs_layout_passes=False)
_SC_PARAMS_BF = pltpu.CompilerParams(needs_layout_passes=False,
                                     use_tc_tiling_on_sc=False)


def _deg_body(src_hbm, dst_hbm, z_hbm, out_hbm, sidx, didx, hs, hd):
    c = lax.axis_index("c")
    s = lax.axis_index("s")
    w = s * NC + c
    pltpu.sync_copy(src_hbm.at[w], sidx)
    pltpu.sync_copy(dst_hbm.at[w], didx)
    pltpu.sync_copy(z_hbm, hs)
    pltpu.sync_copy(z_hbm, hd)
    ones = jnp.ones((16,), jnp.float32)

    def body(i, carry):
        si = sidx[pl.ds(i * 16, 16)]
        di = didx[pl.ds(i * 16, 16)]
        plsc.addupdate_scatter(hs, [si], ones)
        plsc.addupdate_scatter(hd, [di], ones)
        return carry

    lax.fori_loop(0, EPT // 16, body, 0)
    pltpu.sync_copy(hs, out_hbm.at[0, w])
    pltpu.sync_copy(hd, out_hbm.at[1, w])


_deg = pl.kernel(
    _deg_body,
    out_type=jax.ShapeDtypeStruct((2, NW, NPAD), jnp.float32),
    mesh=_sc_mesh(),
    scratch_types=[
        pltpu.VMEM((EPT,), jnp.int32),
        pltpu.VMEM((EPT,), jnp.int32),
        pltpu.VMEM((NPAD,), jnp.float32),
        pltpu.VMEM((NPAD,), jnp.float32),
    ],
    compiler_params=_SC_PARAMS,
)


def _seg_body(tab_hbm, src_hbm, dst_hbm, z_hbm, out_hbm, sidx, didx, rows0,
              rows1, acc, gsem0, gsem1):
    c = lax.axis_index("c")
    s = lax.axis_index("s")
    w = s * NC + c
    pltpu.sync_copy(src_hbm.at[w], sidx)
    pltpu.sync_copy(dst_hbm.at[w], didx)
    base = s * RPT
    for k in range(RPT // 128):
        pltpu.sync_copy(z_hbm, acc.at[pl.ds(base + k * 128, 128)])
    plsc.subcore_barrier()

    def body(j, carry):
        b0 = j * 2
        b1 = b0 + 1
        pltpu.async_copy(tab_hbm.at[sidx.at[b0]], rows0, gsem0)
        pltpu.async_copy(tab_hbm.at[sidx.at[b1]], rows1, gsem1)
        pltpu.make_async_copy(tab_hbm.at[sidx.at[b0]], rows0, gsem0).wait()
        pltpu.sync_copy(rows0, acc.at[didx.at[b0]], add=True)
        pltpu.make_async_copy(tab_hbm.at[sidx.at[b1]], rows1, gsem1).wait()
        pltpu.sync_copy(rows1, acc.at[didx.at[b1]], add=True)
        return carry

    lax.fori_loop(0, NB // 2, body, 0)
    plsc.subcore_barrier()
    pltpu.sync_copy(acc.at[pl.ds(base, RPT)], out_hbm.at[c, pl.ds(base, RPT)])


_segsum = pl.kernel(
    _seg_body,
    out_type=jax.ShapeDtypeStruct((NC, NPAD, D), jnp.bfloat16),
    mesh=_sc_mesh(),
    scratch_types=[
        pltpu.VMEM((NB, 128), jnp.int32),
        pltpu.VMEM((NB, 128), jnp.int32),
        pltpu.VMEM((128, D), jnp.bfloat16),
        pltpu.VMEM((128, D), jnp.bfloat16),
        pltpu.VMEM_SHARED((NPAD, D), jnp.bfloat16),
        pltpu.SemaphoreType.DMA,
        pltpu.SemaphoreType.DMA,
    ],
    compiler_params=_SC_PARAMS_BF,
)


def _col_bcast(deg_nw_blk):
    return lax.dot_general(
        deg_nw_blk, jnp.ones((NW, 128), jnp.float32),
        (((0,), (0,)), ((), ())), preferred_element_type=jnp.float32)


def _rsq(counts):
    return lax.rsqrt(jnp.maximum(counts, 1.0))


def _l1_body(x_ref, w_ref, dg_ref, t0_ref):
    rout = _rsq(_col_bcast(dg_ref[0]))
    t0_ref[...] = jnp.dot(x_ref[...] * rout, w_ref[...],
                          preferred_element_type=jnp.float32
                          ).astype(jnp.bfloat16)


_layer1 = pl.pallas_call(
    _l1_body,
    grid=(GRID,),
    in_specs=[
        pl.BlockSpec((BLK, D), lambda i: (i, 0)),
        pl.BlockSpec((D, D), lambda i: (0, 0)),
        pl.BlockSpec((1, NW, BLK), lambda i: (0, 0, i)),
    ],
    out_specs=pl.BlockSpec((BLK, D), lambda i: (i, 0)),
    out_shape=jax.ShapeDtypeStruct((NPAD, D), jnp.bfloat16),
)


def _l2_body(p_ref, dg_ref, b0_ref, w_ref, t1_ref):
    agg = p_ref[0].astype(jnp.float32) + p_ref[1].astype(jnp.float32)
    rin = _rsq(_col_bcast(dg_ref[1]))
    rout = _rsq(_col_bcast(dg_ref[0]))
    h = jnp.maximum(agg * rin + b0_ref[...], 0.0)
    t1_ref[...] = jnp.dot(h * rout, w_ref[...],
                          preferred_element_type=jnp.float32
                          ).astype(jnp.bfloat16)


_layer2 = pl.pallas_call(
    _l2_body,
    grid=(GRID,),
    in_specs=[
        pl.BlockSpec((NC, BLK, D), lambda i: (0, i, 0)),
        pl.BlockSpec((2, NW, BLK), lambda i: (0, 0, i)),
        pl.BlockSpec((1, D), lambda i: (0, 0)),
        pl.BlockSpec((D, D), lambda i: (0, 0)),
    ],
    out_specs=pl.BlockSpec((BLK, D), lambda i: (i, 0)),
    out_shape=jax.ShapeDtypeStruct((NPAD, D), jnp.bfloat16),
)


def _fin_body(p_ref, dg_ref, b1_ref, o_ref, acc_ref, sn_ref):
    i = pl.program_id(0)

    @pl.when(i == 0)
    def _init():
        acc_ref[...] = jnp.zeros_like(acc_ref)
        sn_ref[0] = 0.0

    agg = p_ref[0].astype(jnp.float32) + p_ref[1].astype(jnp.float32)
    rin = _rsq(_col_bcast(dg_ref[0]))
    h = agg * rin + b1_ref[...]
    rowid = lax.broadcasted_iota(jnp.int32, (BLK, D), 0) + i * BLK
    h = jnp.where(rowid < N_NODES, h, 0.0)
    acc_ref[...] += jnp.sum(h, axis=0, keepdims=True)
    sn_ref[0] += jnp.sum(jnp.sqrt(jnp.sum(h * h, axis=1)))

    @pl.when(i == pl.num_programs(0) - 1)
    def _done():
        factor = jnp.sqrt(jnp.float32(D)) * (jnp.float32(N_NODES) / sn_ref[0])
        o_ref[...] = acc_ref[...] * factor


_final = pl.pallas_call(
    _fin_body,
    grid=(GRID,),
    in_specs=[
        pl.BlockSpec((NC, BLK, D), lambda i: (0, i, 0)),
        pl.BlockSpec((1, NW, BLK), lambda i: (1, 0, i)),
        pl.BlockSpec((1, D), lambda i: (0, 0)),
    ],
    out_specs=pl.BlockSpec((1, D), lambda i: (0, 0)),
    out_shape=jax.ShapeDtypeStruct((1, D), jnp.float32),
    scratch_shapes=[
        pltpu.VMEM((1, D), jnp.float32),
        pltpu.SMEM((1,), jnp.float32),
    ],
)


def kernel(x, edge_index, edge_attr, W0, b0, W1, b1):
    src = edge_index[0]
    dst = edge_index[1]
    padv = N_NODES + (jnp.arange(E_PAD - N_EDGES, dtype=jnp.int32)
                      % (NPAD - N_NODES))
    srcp = jnp.concatenate([src, padv])
    dstp = jnp.concatenate([dst, padv])
    src1 = srcp.reshape(NW, EPT)
    dst1 = dstp.reshape(NW, EPT)
    src3 = srcp.reshape(NW, NB, 128)
    dst3 = dstp.reshape(NW, NB, 128)
    x_pad = jnp.zeros((NPAD, D), jnp.float32).at[:N_NODES].set(x)
    z1 = jnp.zeros((NPAD,), jnp.float32)
    z2 = jnp.zeros((128, D), jnp.bfloat16)
    b0r = b0.reshape(1, D)
    b1r = b1.reshape(1, D)

    degp = _deg(src1, dst1, z1)
    t0 = _layer1(x_pad, W0, degp)
    p0 = _segsum(t0, src3, dst3, z2)
    t1 = _layer2(p0, degp, b0r, W1)
    p1 = _segsum(t1, src3, dst3, z2)
    return _final(p1, degp, b1r)


# speedup vs baseline: 4.1673x; 1.1543x over previous
import jax
import jax.numpy as jnp
from jax import lax
from jax.experimental import pallas as pl
from jax.experimental.pallas import tpu as pltpu
from jax.experimental.pallas import tpu_sc as plsc

N_NODES = 10000
N_EDGES = 320000
D = 128
NPAD = 10240
NC = 2
NS = 16
NW = NC * NS
NB = 80
EPT = NB * 128
E_PAD = NW * EPT
RPT = NPAD // NS
BLK = 1280
GRID = NPAD // BLK


def _sc_mesh():
    return plsc.VectorSubcoreMesh(core_axis_name="c", subcore_axis_name="s")


_SC_PARAMS = pltpu.CompilerParams(needs_layout_passes=False)
_SC_PARAMS_BF = pltpu.CompilerParams(needs_layout_passes=False,
                                     use_tc_tiling_on_sc=False)


def _deg_body(src_hbm, dst_hbm, z_hbm, out_hbm, sidx, didx, hs, hd):
    c = lax.axis_index("c")
    s = lax.axis_index("s")
    w = s * NC + c
    pltpu.sync_copy(src_hbm.at[w], sidx)
    pltpu.sync_copy(dst_hbm.at[w], didx)
    pltpu.sync_copy(z_hbm, hs)
    pltpu.sync_copy(z_hbm, hd)
    ones = jnp.ones((16,), jnp.float32)

    def body(i, carry):
        si = sidx[pl.ds(i * 16, 16)]
        di = didx[pl.ds(i * 16, 16)]
        plsc.addupdate_scatter(hs, [si], ones)
        plsc.addupdate_scatter(hd, [di], ones)
        return carry

    lax.fori_loop(0, EPT // 16, body, 0)
    pltpu.sync_copy(hs, out_hbm.at[0, w])
    pltpu.sync_copy(hd, out_hbm.at[1, w])


_deg = pl.kernel(
    _deg_body,
    out_type=jax.ShapeDtypeStruct((2, NW, NPAD), jnp.float32),
    mesh=_sc_mesh(),
    scratch_types=[
        pltpu.VMEM((EPT,), jnp.int32),
        pltpu.VMEM((EPT,), jnp.int32),
        pltpu.VMEM((NPAD,), jnp.float32),
        pltpu.VMEM((NPAD,), jnp.float32),
    ],
    compiler_params=_SC_PARAMS,
)


def _seg_body(tab_hbm, src_hbm, dst_hbm, z_hbm, out_hbm, sidx, didx, rows0,
              rows1, acc, gsem0, gsem1):
    c = lax.axis_index("c")
    s = lax.axis_index("s")
    w = s * NC + c
    pltpu.sync_copy(src_hbm.at[w], sidx)
    pltpu.sync_copy(dst_hbm.at[w], didx)
    base = s * RPT
    for k in range(RPT // 128):
        pltpu.sync_copy(z_hbm, acc.at[pl.ds(base + k * 128, 128)])
    plsc.subcore_barrier()

    pltpu.async_copy(tab_hbm.at[sidx.at[0]], rows0, gsem0)
    pltpu.async_copy(tab_hbm.at[sidx.at[1]], rows1, gsem1)

    def body(j, carry):
        b0 = j * 2
        b1 = b0 + 1
        pltpu.make_async_copy(tab_hbm.at[sidx.at[b0]], rows0, gsem0).wait()
        pltpu.sync_copy(rows0, acc.at[didx.at[b0]], add=True)
        pltpu.async_copy(tab_hbm.at[sidx.at[b0 + 2]], rows0, gsem0)
        pltpu.make_async_copy(tab_hbm.at[sidx.at[b1]], rows1, gsem1).wait()
        pltpu.sync_copy(rows1, acc.at[didx.at[b1]], add=True)
        pltpu.async_copy(tab_hbm.at[sidx.at[b1 + 2]], rows1, gsem1)
        return carry

    lax.fori_loop(0, NB // 2 - 1, body, 0)
    pltpu.make_async_copy(tab_hbm.at[sidx.at[NB - 2]], rows0, gsem0).wait()
    pltpu.sync_copy(rows0, acc.at[didx.at[NB - 2]], add=True)
    pltpu.make_async_copy(tab_hbm.at[sidx.at[NB - 1]], rows1, gsem1).wait()
    pltpu.sync_copy(rows1, acc.at[didx.at[NB - 1]], add=True)
    plsc.subcore_barrier()
    pltpu.sync_copy(acc.at[pl.ds(base, RPT)], out_hbm.at[c, pl.ds(base, RPT)])


_segsum = pl.kernel(
    _seg_body,
    out_type=jax.ShapeDtypeStruct((NC, NPAD, D), jnp.bfloat16),
    mesh=_sc_mesh(),
    scratch_types=[
        pltpu.VMEM((NB, 128), jnp.int32),
        pltpu.VMEM((NB, 128), jnp.int32),
        pltpu.VMEM((128, D), jnp.bfloat16),
        pltpu.VMEM((128, D), jnp.bfloat16),
        pltpu.VMEM_SHARED((NPAD, D), jnp.bfloat16),
        pltpu.SemaphoreType.DMA,
        pltpu.SemaphoreType.DMA,
    ],
    compiler_params=_SC_PARAMS_BF,
)


def _col_bcast(deg_nw_blk):
    return lax.dot_general(
        deg_nw_blk, jnp.ones((NW, 128), jnp.float32),
        (((0,), (0,)), ((), ())), preferred_element_type=jnp.float32)


def _rsq(counts):
    return lax.rsqrt(jnp.maximum(counts, 1.0))


def _l1_body(x_ref, w_ref, dg_ref, t0_ref):
    rout = _rsq(_col_bcast(dg_ref[0]))
    t0_ref[...] = jnp.dot(x_ref[...] * rout, w_ref[...],
                          preferred_element_type=jnp.float32
                          ).astype(jnp.bfloat16)


_layer1 = pl.pallas_call(
    _l1_body,
    grid=(GRID,),
    in_specs=[
        pl.BlockSpec((BLK, D), lambda i: (i, 0)),
        pl.BlockSpec((D, D), lambda i: (0, 0)),
        pl.BlockSpec((1, NW, BLK), lambda i: (0, 0, i)),
    ],
    out_specs=pl.BlockSpec((BLK, D), lambda i: (i, 0)),
    out_shape=jax.ShapeDtypeStruct((NPAD, D), jnp.bfloat16),
)


def _l2_body(p_ref, dg_ref, b0_ref, w_ref, t1_ref):
    agg = p_ref[0].astype(jnp.float32) + p_ref[1].astype(jnp.float32)
    rin = _rsq(_col_bcast(dg_ref[1]))
    rout = _rsq(_col_bcast(dg_ref[0]))
    h = jnp.maximum(agg * rin + b0_ref[...], 0.0)
    t1_ref[...] = jnp.dot(h * rout, w_ref[...],
                          preferred_element_type=jnp.float32
                          ).astype(jnp.bfloat16)


_layer2 = pl.pallas_call(
    _l2_body,
    grid=(GRID,),
    in_specs=[
        pl.BlockSpec((NC, BLK, D), lambda i: (0, i, 0)),
        pl.BlockSpec((2, NW, BLK), lambda i: (0, 0, i)),
        pl.BlockSpec((1, D), lambda i: (0, 0)),
        pl.BlockSpec((D, D), lambda i: (0, 0)),
    ],
    out_specs=pl.BlockSpec((BLK, D), lambda i: (i, 0)),
    out_shape=jax.ShapeDtypeStruct((NPAD, D), jnp.bfloat16),
)


def _fin_body(p_ref, dg_ref, b1_ref, o_ref, acc_ref, sn_ref):
    i = pl.program_id(0)

    @pl.when(i == 0)
    def _init():
        acc_ref[...] = jnp.zeros_like(acc_ref)
        sn_ref[0] = 0.0

    agg = p_ref[0].astype(jnp.float32) + p_ref[1].astype(jnp.float32)
    rin = _rsq(_col_bcast(dg_ref[0]))
    h = agg * rin + b1_ref[...]
    rowid = lax.broadcasted_iota(jnp.int32, (BLK, D), 0) + i * BLK
    h = jnp.where(rowid < N_NODES, h, 0.0)
    acc_ref[...] += jnp.sum(h, axis=0, keepdims=True)
    sn_ref[0] += jnp.sum(jnp.sqrt(jnp.sum(h * h, axis=1)))

    @pl.when(i == pl.num_programs(0) - 1)
    def _done():
        factor = jnp.sqrt(jnp.float32(D)) * (jnp.float32(N_NODES) / sn_ref[0])
        o_ref[...] = acc_ref[...] * factor


_final = pl.pallas_call(
    _fin_body,
    grid=(GRID,),
    in_specs=[
        pl.BlockSpec((NC, BLK, D), lambda i: (0, i, 0)),
        pl.BlockSpec((1, NW, BLK), lambda i: (1, 0, i)),
        pl.BlockSpec((1, D), lambda i: (0, 0)),
    ],
    out_specs=pl.BlockSpec((1, D), lambda i: (0, 0)),
    out_shape=jax.ShapeDtypeStruct((1, D), jnp.float32),
    scratch_shapes=[
        pltpu.VMEM((1, D), jnp.float32),
        pltpu.SMEM((1,), jnp.float32),
    ],
)


def kernel(x, edge_index, edge_attr, W0, b0, W1, b1):
    src = edge_index[0]
    dst = edge_index[1]
    padv = N_NODES + (jnp.arange(E_PAD - N_EDGES, dtype=jnp.int32)
                      % (NPAD - N_NODES))
    srcp = jnp.concatenate([src, padv])
    dstp = jnp.concatenate([dst, padv])
    src1 = srcp.reshape(NW, EPT)
    dst1 = dstp.reshape(NW, EPT)
    src3 = srcp.reshape(NW, NB, 128)
    dst3 = dstp.reshape(NW, NB, 128)
    x_pad = jnp.zeros((NPAD, D), jnp.float32).at[:N_NODES].set(x)
    z1 = jnp.zeros((NPAD,), jnp.float32)
    z2 = jnp.zeros((128, D), jnp.bfloat16)
    b0r = b0.reshape(1, D)
    b1r = b1.reshape(1, D)

    degp = _deg(src1, dst1, z1)
    t0 = _layer1(x_pad, W0, degp)
    p0 = _segsum(t0, src3, dst3, z2)
    t1 = _layer2(p0, degp, b0r, W1)
    p1 = _segsum(t1, src3, dst3, z2)
    return _final(p1, degp, b1r)


# speedup vs baseline: 4.1997x; 1.0078x over previous
import jax
import jax.numpy as jnp
from jax import lax
from jax.experimental import pallas as pl
from jax.experimental.pallas import tpu as pltpu
from jax.experimental.pallas import tpu_sc as plsc

N_NODES = 10000
N_EDGES = 320000
D = 128
NPAD = 10240
NC = 2
NS = 16
NW = NC * NS
NB = 80
EPT = NB * 128
E_PAD = NW * EPT
RPT = NPAD // NS
BLK = 1280
GRID = NPAD // BLK


def _sc_mesh():
    return plsc.VectorSubcoreMesh(core_axis_name="c", subcore_axis_name="s")


_SC_PARAMS = pltpu.CompilerParams(needs_layout_passes=False)
_SC_PARAMS_BF = pltpu.CompilerParams(needs_layout_passes=False,
                                     use_tc_tiling_on_sc=False)


def _deg_body(src_hbm, dst_hbm, z_hbm, out_hbm, sidx, didx, hs, hd):
    c = lax.axis_index("c")
    s = lax.axis_index("s")
    w = s * NC + c
    pltpu.sync_copy(src_hbm.at[w], sidx)
    pltpu.sync_copy(dst_hbm.at[w], didx)
    pltpu.sync_copy(z_hbm, hs)
    pltpu.sync_copy(z_hbm, hd)
    ones = jnp.ones((16,), jnp.float32)

    def body(i, carry):
        for u in range(4):
            si = sidx[pl.ds(i * 64 + u * 16, 16)]
            di = didx[pl.ds(i * 64 + u * 16, 16)]
            plsc.addupdate_scatter(hs, [si], ones)
            plsc.addupdate_scatter(hd, [di], ones)
        return carry

    lax.fori_loop(0, EPT // 64, body, 0)
    pltpu.sync_copy(hs, out_hbm.at[0, w])
    pltpu.sync_copy(hd, out_hbm.at[1, w])


_deg = pl.kernel(
    _deg_body,
    out_type=jax.ShapeDtypeStruct((2, NW, NPAD), jnp.float32),
    mesh=_sc_mesh(),
    scratch_types=[
        pltpu.VMEM((EPT,), jnp.int32),
        pltpu.VMEM((EPT,), jnp.int32),
        pltpu.VMEM((NPAD,), jnp.float32),
        pltpu.VMEM((NPAD,), jnp.float32),
    ],
    compiler_params=_SC_PARAMS,
)


def _seg_body(tab_hbm, src_hbm, dst_hbm, z_hbm, out_hbm, sidx, didx, rows0,
              rows1, acc, gsem0, gsem1):
    c = lax.axis_index("c")
    s = lax.axis_index("s")
    w = s * NC + c
    pltpu.sync_copy(src_hbm.at[w], sidx)
    pltpu.sync_copy(dst_hbm.at[w], didx)
    base = s * RPT
    for k in range(RPT // 128):
        pltpu.sync_copy(z_hbm, acc.at[pl.ds(base + k * 128, 128)])
    plsc.subcore_barrier()

    pltpu.async_copy(tab_hbm.at[sidx.at[0]], rows0, gsem0)
    pltpu.async_copy(tab_hbm.at[sidx.at[1]], rows1, gsem1)

    def body(j, carry):
        b0 = j * 2
        b1 = b0 + 1
        pltpu.make_async_copy(tab_hbm.at[sidx.at[b0]], rows0, gsem0).wait()
        pltpu.sync_copy(rows0, acc.at[didx.at[b0]], add=True)
        pltpu.async_copy(tab_hbm.at[sidx.at[b0 + 2]], rows0, gsem0)
        pltpu.make_async_copy(tab_hbm.at[sidx.at[b1]], rows1, gsem1).wait()
        pltpu.sync_copy(rows1, acc.at[didx.at[b1]], add=True)
        pltpu.async_copy(tab_hbm.at[sidx.at[b1 + 2]], rows1, gsem1)
        return carry

    lax.fori_loop(0, NB // 2 - 1, body, 0)
    pltpu.make_async_copy(tab_hbm.at[sidx.at[NB - 2]], rows0, gsem0).wait()
    pltpu.sync_copy(rows0, acc.at[didx.at[NB - 2]], add=True)
    pltpu.make_async_copy(tab_hbm.at[sidx.at[NB - 1]], rows1, gsem1).wait()
    pltpu.sync_copy(rows1, acc.at[didx.at[NB - 1]], add=True)
    plsc.subcore_barrier()
    pltpu.sync_copy(acc.at[pl.ds(base, RPT)], out_hbm.at[c, pl.ds(base, RPT)])


_segsum = pl.kernel(
    _seg_body,
    out_type=jax.ShapeDtypeStruct((NC, NPAD, D), jnp.bfloat16),
    mesh=_sc_mesh(),
    scratch_types=[
        pltpu.VMEM((NB, 128), jnp.int32),
        pltpu.VMEM((NB, 128), jnp.int32),
        pltpu.VMEM((128, D), jnp.bfloat16),
        pltpu.VMEM((128, D), jnp.bfloat16),
        pltpu.VMEM_SHARED((NPAD, D), jnp.bfloat16),
        pltpu.SemaphoreType.DMA,
        pltpu.SemaphoreType.DMA,
    ],
    compiler_params=_SC_PARAMS_BF,
)


def _col_bcast(deg_nw_blk):
    return lax.dot_general(
        deg_nw_blk, jnp.ones((NW, 128), jnp.float32),
        (((0,), (0,)), ((), ())), preferred_element_type=jnp.float32)


def _rsq(counts):
    return lax.rsqrt(jnp.maximum(counts, 1.0))


def _l1_body(x_ref, w_ref, dg_ref, t0_ref):
    rout = _rsq(_col_bcast(dg_ref[0]))
    t0_ref[...] = jnp.dot(x_ref[...] * rout, w_ref[...],
                          preferred_element_type=jnp.float32
                          ).astype(jnp.bfloat16)


_layer1 = pl.pallas_call(
    _l1_body,
    grid=(GRID,),
    in_specs=[
        pl.BlockSpec((BLK, D), lambda i: (i, 0)),
        pl.BlockSpec((D, D), lambda i: (0, 0)),
        pl.BlockSpec((1, NW, BLK), lambda i: (0, 0, i)),
    ],
    out_specs=pl.BlockSpec((BLK, D), lambda i: (i, 0)),
    out_shape=jax.ShapeDtypeStruct((NPAD, D), jnp.bfloat16),
)


def _l2_body(p_ref, dg_ref, b0_ref, w_ref, t1_ref):
    agg = p_ref[0].astype(jnp.float32) + p_ref[1].astype(jnp.float32)
    rin = _rsq(_col_bcast(dg_ref[1]))
    rout = _rsq(_col_bcast(dg_ref[0]))
    h = jnp.maximum(agg * rin + b0_ref[...], 0.0)
    t1_ref[...] = jnp.dot(h * rout, w_ref[...],
                          preferred_element_type=jnp.float32
                          ).astype(jnp.bfloat16)


_layer2 = pl.pallas_call(
    _l2_body,
    grid=(GRID,),
    in_specs=[
        pl.BlockSpec((NC, BLK, D), lambda i: (0, i, 0)),
        pl.BlockSpec((2, NW, BLK), lambda i: (0, 0, i)),
        pl.BlockSpec((1, D), lambda i: (0, 0)),
        pl.BlockSpec((D, D), lambda i: (0, 0)),
    ],
    out_specs=pl.BlockSpec((BLK, D), lambda i: (i, 0)),
    out_shape=jax.ShapeDtypeStruct((NPAD, D), jnp.bfloat16),
)


def _fin_body(p_ref, dg_ref, b1_ref, o_ref, acc_ref, sn_ref):
    i = pl.program_id(0)

    @pl.when(i == 0)
    def _init():
        acc_ref[...] = jnp.zeros_like(acc_ref)
        sn_ref[0] = 0.0

    agg = p_ref[0].astype(jnp.float32) + p_ref[1].astype(jnp.float32)
    rin = _rsq(_col_bcast(dg_ref[0]))
    h = agg * rin + b1_ref[...]
    rowid = lax.broadcasted_iota(jnp.int32, (BLK, D), 0) + i * BLK
    h = jnp.where(rowid < N_NODES, h, 0.0)
    acc_ref[...] += jnp.sum(h, axis=0, keepdims=True)
    sn_ref[0] += jnp.sum(jnp.sqrt(jnp.sum(h * h, axis=1)))

    @pl.when(i == pl.num_programs(0) - 1)
    def _done():
        factor = jnp.sqrt(jnp.float32(D)) * (jnp.float32(N_NODES) / sn_ref[0])
        o_ref[...] = acc_ref[...] * factor


_final = pl.pallas_call(
    _fin_body,
    grid=(GRID,),
    in_specs=[
        pl.BlockSpec((NC, BLK, D), lambda i: (0, i, 0)),
        pl.BlockSpec((1, NW, BLK), lambda i: (1, 0, i)),
        pl.BlockSpec((1, D), lambda i: (0, 0)),
    ],
    out_specs=pl.BlockSpec((1, D), lambda i: (0, 0)),
    out_shape=jax.ShapeDtypeStruct((1, D), jnp.float32),
    scratch_shapes=[
        pltpu.VMEM((1, D), jnp.float32),
        pltpu.SMEM((1,), jnp.float32),
    ],
)


def kernel(x, edge_index, edge_attr, W0, b0, W1, b1):
    src = edge_index[0]
    dst = edge_index[1]
    padv = N_NODES + (jnp.arange(E_PAD - N_EDGES, dtype=jnp.int32)
                      % (NPAD - N_NODES))
    srcp = jnp.concatenate([src, padv])
    dstp = jnp.concatenate([dst, padv])
    src1 = srcp.reshape(NW, EPT)
    dst1 = dstp.reshape(NW, EPT)
    src3 = srcp.reshape(NW, NB, 128)
    dst3 = dstp.reshape(NW, NB, 128)
    z1 = jnp.zeros((NPAD,), jnp.float32)
    z2 = jnp.zeros((128, D), jnp.bfloat16)
    b0r = b0.reshape(1, D)
    b1r = b1.reshape(1, D)

    degp = _deg(src1, dst1, z1)
    t0 = _layer1(x, W0, degp)
    p0 = _segsum(t0, src3, dst3, z2)
    t1 = _layer2(p0, degp, b0r, W1)
    p1 = _segsum(t1, src3, dst3, z2)
    return _final(p1, degp, b1r)
